# Initial kernel scaffold; baseline (speedup 1.0000x reference)
import jax
import jax.numpy as jnp
from jax.experimental import pallas as pl


def kernel(input_feat, delta, gate_W, gate_b, expert_W, expert_b):
    raise NotImplementedError("write your pallas kernel here")


# speedup vs baseline: 131.5488x; 131.5488x over previous
import jax
import jax.numpy as jnp
from jax.experimental import pallas as pl

TOPK = 2


def _moe_kernel(x_ref, d_ref, gw_ref, gb_ref, ew_ref, eb_ref, o_ref):
    blk = x_ref.shape[0]
    E = gw_ref.shape[1]
    logits = jnp.dot(x_ref[:], gw_ref[:], preferred_element_type=jnp.float32)
    logits = logits + gb_ref[0][None, :]
    e_iota = jax.lax.broadcasted_iota(jnp.int32, (blk, E), 1)
    m1 = jnp.max(logits, axis=1, keepdims=True)
    i1 = jnp.min(jnp.where(logits == m1, e_iota, E), axis=1, keepdims=True)
    masked = jnp.where(e_iota == i1, -jnp.inf, logits)
    m2 = jnp.max(masked, axis=1, keepdims=True)
    i2 = jnp.min(jnp.where(masked == m2, e_iota, E), axis=1, keepdims=True)
    b = jnp.exp(m2 - m1)
    w1 = 1.0 / (1.0 + b)
    w2 = b * w1
    w = jnp.where(e_iota == i1, w1, 0.0) + jnp.where(e_iota == i2, w2, 0.0)

    delta = d_ref[:]
    acc = jnp.dot(w, eb_ref[:], preferred_element_type=jnp.float32)
    for e in range(E):
        acc = acc + w[:, e:e + 1] * jnp.dot(
            delta, ew_ref[e], preferred_element_type=jnp.float32)
    o_ref[:] = acc


def kernel(input_feat, delta, gate_W, gate_b, expert_W, expert_b):
    B, T, D = input_feat.shape
    E = gate_W.shape[1]
    N = B * T
    x = input_feat.reshape(N, D)
    d = delta.reshape(N, D)
    gb = gate_b.reshape(1, E)

    BLK = 1024
    grid = (N // BLK,)
    out = pl.pallas_call(
        _moe_kernel,
        grid=grid,
        in_specs=[
            pl.BlockSpec((BLK, D), lambda i: (i, 0)),
            pl.BlockSpec((BLK, D), lambda i: (i, 0)),
            pl.BlockSpec((D, E), lambda i: (0, 0)),
            pl.BlockSpec((1, E), lambda i: (0, 0)),
            pl.BlockSpec((E, D, D), lambda i: (0, 0, 0)),
            pl.BlockSpec((E, D), lambda i: (0, 0)),
        ],
        out_specs=pl.BlockSpec((BLK, D), lambda i: (i, 0)),
        out_shape=jax.ShapeDtypeStruct((N, D), jnp.float32),
    )(x, d, gate_W, gb, expert_W, expert_b)
    return out.reshape(B, T, D)


# speedup vs baseline: 131.7584x; 1.0016x over previous
import jax
import jax.numpy as jnp
from jax.experimental import pallas as pl

TOPK = 2


def _moe_kernel(x_ref, d_ref, gw_ref, gb_ref, ew_ref, eb_ref, o_ref):
    blk = x_ref.shape[0]
    E = gw_ref.shape[1]
    logits = jnp.dot(x_ref[:], gw_ref[:], preferred_element_type=jnp.float32)
    logits = logits + gb_ref[0][None, :]
    e_iota = jax.lax.broadcasted_iota(jnp.int32, (blk, E), 1)
    m1 = jnp.max(logits, axis=1, keepdims=True)
    i1 = jnp.min(jnp.where(logits == m1, e_iota, E), axis=1, keepdims=True)
    masked = jnp.where(e_iota == i1, -jnp.inf, logits)
    m2 = jnp.max(masked, axis=1, keepdims=True)
    i2 = jnp.min(jnp.where(masked == m2, e_iota, E), axis=1, keepdims=True)
    b = jnp.exp(m2 - m1)
    w1 = 1.0 / (1.0 + b)
    w2 = b * w1
    w = jnp.where(e_iota == i1, w1, 0.0) + jnp.where(e_iota == i2, w2, 0.0)

    delta = d_ref[:].astype(jnp.bfloat16)
    acc = jnp.dot(w, eb_ref[:], preferred_element_type=jnp.float32)
    for e in range(E):
        acc = acc + w[:, e:e + 1] * jnp.dot(
            delta, ew_ref[e].astype(jnp.bfloat16),
            preferred_element_type=jnp.float32)
    o_ref[:] = acc


def kernel(input_feat, delta, gate_W, gate_b, expert_W, expert_b):
    B, T, D = input_feat.shape
    E = gate_W.shape[1]
    N = B * T
    x = input_feat.reshape(N, D)
    d = delta.reshape(N, D)
    gb = gate_b.reshape(1, E)

    BLK = 1024
    grid = (N // BLK,)
    out = pl.pallas_call(
        _moe_kernel,
        grid=grid,
        in_specs=[
            pl.BlockSpec((BLK, D), lambda i: (i, 0)),
            pl.BlockSpec((BLK, D), lambda i: (i, 0)),
            pl.BlockSpec((D, E), lambda i: (0, 0)),
            pl.BlockSpec((1, E), lambda i: (0, 0)),
            pl.BlockSpec((E, D, D), lambda i: (0, 0, 0)),
            pl.BlockSpec((E, D), lambda i: (0, 0)),
        ],
        out_specs=pl.BlockSpec((BLK, D), lambda i: (i, 0)),
        out_shape=jax.ShapeDtypeStruct((N, D), jnp.float32),
    )(x, d, gate_W, gb, expert_W, expert_b)
    return out.reshape(B, T, D)
